# baseline (device time: 389448 ns/iter reference)
import jax
import jax.numpy as jnp
from jax import lax
from jax.experimental import pallas as pl
from jax.experimental.pallas import tpu as pltpu

N_DEV = 4


def kernel(x, W):
    T = x.shape[0]
    V = W.shape[1]
    V2 = V // 2
    CH = 1024
    logits = jnp.dot(x, W, preferred_element_type=jnp.float32)

    def body(l_ref, dummy_ref, out_ref, comm_cw, comm_ccw, stage, sbuf,
             cw_send, cw_recv, ccw_send, ccw_recv,
             s_send, s_recv, stage_sems, in_sems, cw_credit, ccw_credit):
        my = lax.axis_index("i")
        left = (my + N_DEV - 1) % N_DEV
        right = (my + 1) % N_DEV
        diag = (my + 2) % N_DEV

        barrier = pltpu.get_barrier_semaphore()
        for nbr in (left, right, diag):
            pl.semaphore_signal(barrier, inc=1, device_id=(nbr,),
                                device_id_type=pl.DeviceIdType.MESH)
        pl.semaphore_wait(barrier, 3)

        in_cw = pltpu.make_async_copy(
            l_ref.at[:, 0:V2], comm_cw.at[0], in_sems.at[0])
        in_ccw = pltpu.make_async_copy(
            l_ref.at[:, V2:V], comm_ccw.at[0], in_sems.at[1])
        in_cw.start()
        in_ccw.start()

        pending = {0: None, 1: None}
        pctr = [0]

        def stream_out(src_ref, src_col, ncols, out_base, m_col, inv):
            for c in range(0, ncols, CH):
                p = pctr[0] % 2
                pctr[0] += 1
                if pending[p] is not None:
                    pending[p].wait()
                stage[p] = (jnp.exp(src_ref[:, src_col + c:src_col + c + CH]
                                    - m_col) * inv)
                cp = pltpu.make_async_copy(
                    stage.at[p], out_ref.at[:, pl.ds(out_base + c, CH)],
                    stage_sems.at[p])
                cp.start()
                pending[p] = cp

        stats_state = {}
        for h in range(N_DEV - 1):
            ss, rs = h % 2, (h + 1) % 2
            if h == 2:
                pl.semaphore_wait(cw_credit, 1)
                pl.semaphore_wait(ccw_credit, 1)
            dcw = pltpu.make_async_remote_copy(
                src_ref=comm_cw.at[ss], dst_ref=comm_cw.at[rs],
                send_sem=cw_send.at[ss], recv_sem=cw_recv.at[rs],
                device_id=(right,), device_id_type=pl.DeviceIdType.MESH)
            dccw = pltpu.make_async_remote_copy(
                src_ref=comm_ccw.at[ss], dst_ref=comm_ccw.at[rs],
                send_sem=ccw_send.at[ss], recv_sem=ccw_recv.at[rs],
                device_id=(left,), device_id_type=pl.DeviceIdType.MESH)
            if h == 0:
                in_cw.wait()
            dcw.start()
            if h == 0:
                in_ccw.wait()
            dccw.start()

            if h == 0:
                m_loc = jnp.max(comm_cw[0, :, 0:CH], axis=1, keepdims=True)
                for half in (comm_cw, comm_ccw):
                    for c in range(0, V2, CH):
                        if half is comm_cw and c == 0:
                            continue
                        m_loc = jnp.maximum(m_loc, jnp.max(
                            half[0, :, c:c + CH], axis=1, keepdims=True))
                s_loc = jnp.zeros((T, 1), jnp.float32)
                for half in (comm_cw, comm_ccw):
                    for c in range(0, V2, CH):
                        s_loc = s_loc + jnp.sum(
                            jnp.exp(half[0, :, c:c + CH] - m_loc),
                            axis=1, keepdims=True)
                mg = jnp.broadcast_to(m_loc, (T, 64))
                sg = jnp.broadcast_to(s_loc, (T, 64))
                sbuf[3] = jnp.concatenate([mg, sg], axis=1)

                stats_rdmas = []
                for slot, tgt in ((0, right), (1, left), (2, diag)):
                    d = pltpu.make_async_remote_copy(
                        src_ref=sbuf.at[3], dst_ref=sbuf.at[slot],
                        send_sem=s_send.at[slot], recv_sem=s_recv.at[slot],
                        device_id=(tgt,),
                        device_id_type=pl.DeviceIdType.MESH)
                    d.start()
                    stats_rdmas.append(d)
                for d in stats_rdmas:
                    d.wait()
                for slot in range(3):
                    rm = sbuf[slot, :, 0:64]
                    rsum = sbuf[slot, :, 64:128]
                    m_new = jnp.maximum(mg, rm)
                    sg = sg * jnp.exp(mg - m_new) + rsum * jnp.exp(rm - m_new)
                    mg = m_new
                m_col = mg[:, 0:1]
                inv = 1.0 / sg[:, 0:1]
                stats_state["m"] = m_col
                stats_state["inv"] = inv

                stream_out(comm_cw.at[0], 0, V2, my * V, m_col, inv)
                stream_out(comm_ccw.at[0], 0, V2, my * V + V2, m_col, inv)
            else:
                m_col, inv = stats_state["m"], stats_state["inv"]
                o_cw = (my + N_DEV - h) % N_DEV
                stream_out(comm_cw.at[ss], 0, V2, o_cw * V, m_col, inv)
                o_ccw = (my + h) % N_DEV
                stream_out(comm_ccw.at[ss], 0, V2, o_ccw * V + V2,
                           m_col, inv)

            dcw.wait()
            dccw.wait()
            if h == 1:
                pl.semaphore_signal(cw_credit, inc=1, device_id=(left,),
                                    device_id_type=pl.DeviceIdType.MESH)
                pl.semaphore_signal(ccw_credit, inc=1, device_id=(right,),
                                    device_id_type=pl.DeviceIdType.MESH)

        m_col, inv = stats_state["m"], stats_state["inv"]
        stream_out(comm_cw.at[1], 0, V2, ((my + 1) % N_DEV) * V, m_col, inv)
        stream_out(comm_ccw.at[1], 0, V2, ((my + 3) % N_DEV) * V + V2,
                   m_col, inv)
        for p in (0, 1):
            if pending[p] is not None:
                pending[p].wait()

    dummy = jnp.zeros((T, N_DEV * V), jnp.float32)
    return pl.pallas_call(
        body,
        out_shape=jax.ShapeDtypeStruct((T, N_DEV * V), jnp.float32),
        in_specs=[pl.BlockSpec(memory_space=pl.ANY),
                  pl.BlockSpec(memory_space=pl.ANY)],
        out_specs=pl.BlockSpec(memory_space=pl.ANY),
        input_output_aliases={1: 0},
        scratch_shapes=[
            pltpu.VMEM((2, T, V2), jnp.float32),
            pltpu.VMEM((2, T, V2), jnp.float32),
            pltpu.VMEM((2, T, CH), jnp.float32),
            pltpu.VMEM((N_DEV, T, 128), jnp.float32),
            pltpu.SemaphoreType.DMA((2,)),
            pltpu.SemaphoreType.DMA((2,)),
            pltpu.SemaphoreType.DMA((2,)),
            pltpu.SemaphoreType.DMA((2,)),
            pltpu.SemaphoreType.DMA((3,)),
            pltpu.SemaphoreType.DMA((3,)),
            pltpu.SemaphoreType.DMA((2,)),
            pltpu.SemaphoreType.DMA((2,)),
            pltpu.SemaphoreType.REGULAR,
            pltpu.SemaphoreType.REGULAR,
        ],
        compiler_params=pltpu.CompilerParams(
            collective_id=0, vmem_limit_bytes=63 * 1024 * 1024),
    )(logits, dummy)


# device time: 234177 ns/iter; 1.6630x vs baseline; 1.6630x over previous
import jax
import jax.numpy as jnp
from jax import lax
from jax.experimental import pallas as pl
from jax.experimental.pallas import tpu as pltpu

N_DEV = 4


def kernel(x, W):
    T = x.shape[0]
    V = W.shape[1]
    V2 = V // 2
    CH = 1024
    logits = jnp.dot(x, W, preferred_element_type=jnp.float32)

    def body(l_ref, out_ref, comm_cw, comm_ccw, stage, sbuf,
             cw_send, cw_recv, ccw_send, ccw_recv,
             s_send, s_recv, stage_sems, cw_credit, ccw_credit):
        my = lax.axis_index("i")
        left = (my + N_DEV - 1) % N_DEV
        right = (my + 1) % N_DEV
        diag = (my + 2) % N_DEV

        barrier = pltpu.get_barrier_semaphore()
        for nbr in (left, right, diag):
            pl.semaphore_signal(barrier, inc=1, device_id=(nbr,),
                                device_id_type=pl.DeviceIdType.MESH)
        pl.semaphore_wait(barrier, 3)

        for c in range(0, V2, CH):
            comm_cw[0, :, c:c + CH] = l_ref[:, c:c + CH].astype(jnp.bfloat16)
            comm_ccw[0, :, c:c + CH] = (
                l_ref[:, V2 + c:V2 + c + CH].astype(jnp.bfloat16))

        pending = {0: None, 1: None}
        pctr = [0]

        def stream_out(src_ref, src_col, ncols, out_base, m_col, inv):
            for c in range(0, ncols, CH):
                p = pctr[0] % 2
                pctr[0] += 1
                if pending[p] is not None:
                    pending[p].wait()
                stage[p] = (jnp.exp(src_ref[:, src_col + c:src_col + c + CH]
                                    - m_col) * inv)
                cp = pltpu.make_async_copy(
                    stage.at[p], out_ref.at[:, pl.ds(out_base + c, CH)],
                    stage_sems.at[p])
                cp.start()
                pending[p] = cp

        stats_state = {}
        for h in range(N_DEV - 1):
            ss, rs = h % 2, (h + 1) % 2
            if h == 2:
                pl.semaphore_wait(cw_credit, 1)
                pl.semaphore_wait(ccw_credit, 1)
            dcw = pltpu.make_async_remote_copy(
                src_ref=comm_cw.at[ss], dst_ref=comm_cw.at[rs],
                send_sem=cw_send.at[ss], recv_sem=cw_recv.at[rs],
                device_id=(right,), device_id_type=pl.DeviceIdType.MESH)
            dccw = pltpu.make_async_remote_copy(
                src_ref=comm_ccw.at[ss], dst_ref=comm_ccw.at[rs],
                send_sem=ccw_send.at[ss], recv_sem=ccw_recv.at[rs],
                device_id=(left,), device_id_type=pl.DeviceIdType.MESH)
            dcw.start()
            dccw.start()

            if h == 0:
                m_loc = jnp.max(l_ref[:, 0:CH], axis=1, keepdims=True)
                for c in range(CH, V, CH):
                    m_loc = jnp.maximum(m_loc, jnp.max(
                        l_ref[:, c:c + CH], axis=1, keepdims=True))
                s_loc = jnp.zeros((T, 1), jnp.float32)
                for c in range(0, V, CH):
                    s_loc = s_loc + jnp.sum(
                        jnp.exp(l_ref[:, c:c + CH] - m_loc),
                        axis=1, keepdims=True)
                mg = jnp.broadcast_to(m_loc, (T, 64))
                sg = jnp.broadcast_to(s_loc, (T, 64))
                sbuf[3] = jnp.concatenate([mg, sg], axis=1)

                stats_rdmas = []
                for slot, tgt in ((0, right), (1, left), (2, diag)):
                    d = pltpu.make_async_remote_copy(
                        src_ref=sbuf.at[3], dst_ref=sbuf.at[slot],
                        send_sem=s_send.at[slot], recv_sem=s_recv.at[slot],
                        device_id=(tgt,),
                        device_id_type=pl.DeviceIdType.MESH)
                    d.start()
                    stats_rdmas.append(d)
                for d in stats_rdmas:
                    d.wait()
                for slot in range(3):
                    rm = sbuf[slot, :, 0:64]
                    rsum = sbuf[slot, :, 64:128]
                    m_new = jnp.maximum(mg, rm)
                    sg = sg * jnp.exp(mg - m_new) + rsum * jnp.exp(rm - m_new)
                    mg = m_new
                m_col = mg[:, 0:1]
                inv = 1.0 / sg[:, 0:1]
                stats_state["m"] = m_col
                stats_state["inv"] = inv

                stream_out(l_ref, 0, V, my * V, m_col, inv)
            else:
                m_col, inv = stats_state["m"], stats_state["inv"]
                o_cw = (my + N_DEV - h) % N_DEV
                stream_out(comm_cw.at[ss], 0, V2, o_cw * V, m_col, inv)
                o_ccw = (my + h) % N_DEV
                stream_out(comm_ccw.at[ss], 0, V2, o_ccw * V + V2,
                           m_col, inv)

            dcw.wait()
            dccw.wait()
            if h == 1:
                pl.semaphore_signal(cw_credit, inc=1, device_id=(left,),
                                    device_id_type=pl.DeviceIdType.MESH)
                pl.semaphore_signal(ccw_credit, inc=1, device_id=(right,),
                                    device_id_type=pl.DeviceIdType.MESH)

        m_col, inv = stats_state["m"], stats_state["inv"]
        stream_out(comm_cw.at[1], 0, V2, ((my + 1) % N_DEV) * V, m_col, inv)
        stream_out(comm_ccw.at[1], 0, V2, ((my + 3) % N_DEV) * V + V2,
                   m_col, inv)
        for p in (0, 1):
            if pending[p] is not None:
                pending[p].wait()

    return pl.pallas_call(
        body,
        out_shape=jax.ShapeDtypeStruct((T, N_DEV * V), jnp.float32),
        in_specs=[pl.BlockSpec(memory_space=pltpu.VMEM)],
        out_specs=pl.BlockSpec(memory_space=pl.ANY),
        scratch_shapes=[
            pltpu.VMEM((2, T, V2), jnp.bfloat16),
            pltpu.VMEM((2, T, V2), jnp.bfloat16),
            pltpu.VMEM((2, T, CH), jnp.float32),
            pltpu.VMEM((N_DEV, T, 128), jnp.float32),
            pltpu.SemaphoreType.DMA((2,)),
            pltpu.SemaphoreType.DMA((2,)),
            pltpu.SemaphoreType.DMA((2,)),
            pltpu.SemaphoreType.DMA((2,)),
            pltpu.SemaphoreType.DMA((3,)),
            pltpu.SemaphoreType.DMA((3,)),
            pltpu.SemaphoreType.DMA((2,)),
            pltpu.SemaphoreType.REGULAR,
            pltpu.SemaphoreType.REGULAR,
        ],
        compiler_params=pltpu.CompilerParams(
            collective_id=0, vmem_limit_bytes=63 * 1024 * 1024),
    )(logits)


# device time: 230669 ns/iter; 1.6883x vs baseline; 1.0152x over previous
import jax
import jax.numpy as jnp
from jax import lax
from jax.experimental import pallas as pl
from jax.experimental.pallas import tpu as pltpu

N_DEV = 4


def kernel(x, W):
    T = x.shape[0]
    V = W.shape[1]
    V2 = V // 2
    CH = 1024
    logits = jnp.dot(x, W, preferred_element_type=jnp.float32)

    def body(l_ref, out_ref, comm_cw, comm_ccw, stage, sbuf,
             cw_send, cw_recv, ccw_send, ccw_recv,
             s_send, s_recv, stage_sems, cw_credit, ccw_credit):
        my = lax.axis_index("i")
        left = (my + N_DEV - 1) % N_DEV
        right = (my + 1) % N_DEV
        diag = (my + 2) % N_DEV

        barrier = pltpu.get_barrier_semaphore()
        for nbr in (left, right, diag):
            pl.semaphore_signal(barrier, inc=1, device_id=(nbr,),
                                device_id_type=pl.DeviceIdType.MESH)
        pl.semaphore_wait(barrier, 3)

        for c in range(0, V2, CH):
            comm_cw[0, :, c:c + CH] = l_ref[:, c:c + CH].astype(jnp.bfloat16)
            comm_ccw[0, :, c:c + CH] = (
                l_ref[:, V2 + c:V2 + c + CH].astype(jnp.bfloat16))

        pending = {0: None, 1: None}
        pctr = [0]

        def stream_out(src_ref, src_col, ncols, out_base, m_col, inv):
            for c in range(0, ncols, CH):
                p = pctr[0] % 2
                pctr[0] += 1
                if pending[p] is not None:
                    pending[p].wait()
                stage[p] = (jnp.exp(src_ref[:, src_col + c:src_col + c + CH]
                                    - m_col) * inv)
                cp = pltpu.make_async_copy(
                    stage.at[p], out_ref.at[:, pl.ds(out_base + c, CH)],
                    stage_sems.at[p])
                cp.start()
                pending[p] = cp

        stats_state = {}
        for h in range(2):
            ss, rs = h % 2, (h + 1) % 2
            dcw = pltpu.make_async_remote_copy(
                src_ref=comm_cw.at[ss], dst_ref=comm_cw.at[rs],
                send_sem=cw_send.at[ss], recv_sem=cw_recv.at[rs],
                device_id=(right,), device_id_type=pl.DeviceIdType.MESH)
            dccw = pltpu.make_async_remote_copy(
                src_ref=comm_ccw.at[ss], dst_ref=comm_ccw.at[rs],
                send_sem=ccw_send.at[ss], recv_sem=ccw_recv.at[rs],
                device_id=(left,), device_id_type=pl.DeviceIdType.MESH)
            dcw.start()
            dccw.start()

            if h == 0:
                m_loc = jnp.max(l_ref[:, 0:CH], axis=1, keepdims=True)
                for c in range(CH, V, CH):
                    m_loc = jnp.maximum(m_loc, jnp.max(
                        l_ref[:, c:c + CH], axis=1, keepdims=True))
                s_loc = jnp.zeros((T, 1), jnp.float32)
                for c in range(0, V, CH):
                    s_loc = s_loc + jnp.sum(
                        jnp.exp(l_ref[:, c:c + CH] - m_loc),
                        axis=1, keepdims=True)
                mg = jnp.broadcast_to(m_loc, (T, 64))
                sg = jnp.broadcast_to(s_loc, (T, 64))
                sbuf[3] = jnp.concatenate([mg, sg], axis=1)

                stats_rdmas = []
                for slot, tgt in ((0, right), (1, left), (2, diag)):
                    d = pltpu.make_async_remote_copy(
                        src_ref=sbuf.at[3], dst_ref=sbuf.at[slot],
                        send_sem=s_send.at[slot], recv_sem=s_recv.at[slot],
                        device_id=(tgt,),
                        device_id_type=pl.DeviceIdType.MESH)
                    d.start()
                    stats_rdmas.append(d)
                for d in stats_rdmas:
                    d.wait()
                for slot in range(3):
                    rm = sbuf[slot, :, 0:64]
                    rsum = sbuf[slot, :, 64:128]
                    m_new = jnp.maximum(mg, rm)
                    sg = sg * jnp.exp(mg - m_new) + rsum * jnp.exp(rm - m_new)
                    mg = m_new
                m_col = mg[:, 0:1]
                inv = 1.0 / sg[:, 0:1]
                stats_state["m"] = m_col
                stats_state["inv"] = inv

                stream_out(l_ref, 0, V, my * V, m_col, inv)
            else:
                m_col, inv = stats_state["m"], stats_state["inv"]
                o_cw = (my + N_DEV - h) % N_DEV
                stream_out(comm_cw.at[ss], 0, V2, o_cw * V, m_col, inv)
                o_ccw = (my + h) % N_DEV
                stream_out(comm_ccw.at[ss], 0, V2, o_ccw * V + V2,
                           m_col, inv)

            dcw.wait()
            dccw.wait()
            if h == 1:
                pl.semaphore_signal(cw_credit, inc=1, device_id=(left,),
                                    device_id_type=pl.DeviceIdType.MESH)
                pl.semaphore_signal(ccw_credit, inc=1, device_id=(right,),
                                    device_id_type=pl.DeviceIdType.MESH)

        V4 = V2 // 2
        pl.semaphore_wait(cw_credit, 1)
        pl.semaphore_wait(ccw_credit, 1)
        d2 = []
        for commbuf, sends, recvs, tgt in (
                (comm_cw, cw_send, cw_recv, right),
                (comm_ccw, ccw_send, ccw_recv, left)):
            for sub, (si, ri) in enumerate(((0, 1), (2, 2))):
                d = pltpu.make_async_remote_copy(
                    src_ref=commbuf.at[0, :, pl.ds(sub * V4, V4)],
                    dst_ref=commbuf.at[1, :, pl.ds(sub * V4, V4)],
                    send_sem=sends.at[si], recv_sem=recvs.at[ri],
                    device_id=(tgt,), device_id_type=pl.DeviceIdType.MESH)
                d.start()
                d2.append(d)
        dcw_a, dcw_b, dccw_a, dccw_b = d2

        m_col, inv = stats_state["m"], stats_state["inv"]
        stream_out(comm_cw.at[0], 0, V2, ((my + 2) % N_DEV) * V, m_col, inv)
        stream_out(comm_ccw.at[0], 0, V2, ((my + 2) % N_DEV) * V + V2,
                   m_col, inv)

        o_cw = ((my + 1) % N_DEV) * V
        o_ccw = ((my + 3) % N_DEV) * V + V2
        dcw_a.wait()
        stream_out(comm_cw.at[1], 0, V4, o_cw, m_col, inv)
        dccw_a.wait()
        stream_out(comm_ccw.at[1], 0, V4, o_ccw, m_col, inv)
        dcw_b.wait()
        stream_out(comm_cw.at[1], V4, V4, o_cw + V4, m_col, inv)
        dccw_b.wait()
        stream_out(comm_ccw.at[1], V4, V4, o_ccw + V4, m_col, inv)
        for p in (0, 1):
            if pending[p] is not None:
                pending[p].wait()

    return pl.pallas_call(
        body,
        out_shape=jax.ShapeDtypeStruct((T, N_DEV * V), jnp.float32),
        in_specs=[pl.BlockSpec(memory_space=pltpu.VMEM)],
        out_specs=pl.BlockSpec(memory_space=pl.ANY),
        scratch_shapes=[
            pltpu.VMEM((2, T, V2), jnp.bfloat16),
            pltpu.VMEM((2, T, V2), jnp.bfloat16),
            pltpu.VMEM((2, T, CH), jnp.float32),
            pltpu.VMEM((N_DEV, T, 128), jnp.float32),
            pltpu.SemaphoreType.DMA((3,)),
            pltpu.SemaphoreType.DMA((3,)),
            pltpu.SemaphoreType.DMA((3,)),
            pltpu.SemaphoreType.DMA((3,)),
            pltpu.SemaphoreType.DMA((3,)),
            pltpu.SemaphoreType.DMA((3,)),
            pltpu.SemaphoreType.DMA((2,)),
            pltpu.SemaphoreType.REGULAR,
            pltpu.SemaphoreType.REGULAR,
        ],
        compiler_params=pltpu.CompilerParams(
            collective_id=0, vmem_limit_bytes=63 * 1024 * 1024),
    )(logits)


# device time: 224144 ns/iter; 1.7375x vs baseline; 1.0291x over previous
import jax
import jax.numpy as jnp
from jax import lax
from jax.experimental import pallas as pl
from jax.experimental.pallas import tpu as pltpu

N_DEV = 4


def kernel(x, W):
    T = x.shape[0]
    V = W.shape[1]
    V2 = V // 2
    CH = 1024
    logits = jnp.dot(x, W, preferred_element_type=jnp.float32)

    def body(l_ref, out_ref, comm_cw, comm_ccw, stage, sbuf,
             cw_send, cw_recv, ccw_send, ccw_recv,
             s_send, s_recv, stage_sems, cw_credit, ccw_credit):
        my = lax.axis_index("i")
        left = (my + N_DEV - 1) % N_DEV
        right = (my + 1) % N_DEV
        diag = (my + 2) % N_DEV

        barrier = pltpu.get_barrier_semaphore()
        for nbr in (left, right, diag):
            pl.semaphore_signal(barrier, inc=1, device_id=(nbr,),
                                device_id_type=pl.DeviceIdType.MESH)
        pl.semaphore_wait(barrier, 3)

        for c in range(0, V2, CH):
            comm_cw[0, :, c:c + CH] = l_ref[:, c:c + CH].astype(jnp.bfloat16)

        pending = {0: None, 1: None}
        pctr = [0]

        def stream_out(src_ref, src_col, ncols, out_base, m_col, inv):
            for c in range(0, ncols, CH):
                p = pctr[0] % 2
                pctr[0] += 1
                if pending[p] is not None:
                    pending[p].wait()
                stage[p] = (jnp.exp(src_ref[:, src_col + c:src_col + c + CH]
                                    - m_col) * inv)
                cp = pltpu.make_async_copy(
                    stage.at[p], out_ref.at[:, pl.ds(out_base + c, CH)],
                    stage_sems.at[p])
                cp.start()
                pending[p] = cp

        stats_state = {}
        for h in range(2):
            ss, rs = h % 2, (h + 1) % 2
            dcw = pltpu.make_async_remote_copy(
                src_ref=comm_cw.at[ss], dst_ref=comm_cw.at[rs],
                send_sem=cw_send.at[ss], recv_sem=cw_recv.at[rs],
                device_id=(right,), device_id_type=pl.DeviceIdType.MESH)
            dccw = pltpu.make_async_remote_copy(
                src_ref=comm_ccw.at[ss], dst_ref=comm_ccw.at[rs],
                send_sem=ccw_send.at[ss], recv_sem=ccw_recv.at[rs],
                device_id=(left,), device_id_type=pl.DeviceIdType.MESH)
            dcw.start()
            if h == 0:
                for c in range(0, V2, CH):
                    comm_ccw[0, :, c:c + CH] = (
                        l_ref[:, V2 + c:V2 + c + CH].astype(jnp.bfloat16))
            dccw.start()

            if h == 0:
                m_loc = jnp.max(l_ref[:, 0:CH], axis=1, keepdims=True)
                for c in range(CH, V, CH):
                    m_loc = jnp.maximum(m_loc, jnp.max(
                        l_ref[:, c:c + CH], axis=1, keepdims=True))
                s_loc = jnp.zeros((T, 1), jnp.float32)
                for c in range(0, V, CH):
                    s_loc = s_loc + jnp.sum(
                        jnp.exp(l_ref[:, c:c + CH] - m_loc),
                        axis=1, keepdims=True)
                mg = jnp.broadcast_to(m_loc, (T, 64))
                sg = jnp.broadcast_to(s_loc, (T, 64))
                sbuf[3] = jnp.concatenate([mg, sg], axis=1)

                stats_rdmas = []
                for slot, tgt in ((0, right), (1, left), (2, diag)):
                    d = pltpu.make_async_remote_copy(
                        src_ref=sbuf.at[3], dst_ref=sbuf.at[slot],
                        send_sem=s_send.at[slot], recv_sem=s_recv.at[slot],
                        device_id=(tgt,),
                        device_id_type=pl.DeviceIdType.MESH)
                    d.start()
                    stats_rdmas.append(d)
                for d in stats_rdmas:
                    d.wait()
                for slot in range(3):
                    rm = sbuf[slot, :, 0:64]
                    rsum = sbuf[slot, :, 64:128]
                    m_new = jnp.maximum(mg, rm)
                    sg = sg * jnp.exp(mg - m_new) + rsum * jnp.exp(rm - m_new)
                    mg = m_new
                m_col = mg[:, 0:1]
                inv = 1.0 / sg[:, 0:1]
                stats_state["m"] = m_col
                stats_state["inv"] = inv
            else:
                m_col, inv = stats_state["m"], stats_state["inv"]
                o_cw = (my + N_DEV - h) % N_DEV
                stream_out(comm_cw.at[ss], 0, V2, o_cw * V, m_col, inv)
                o_ccw = (my + h) % N_DEV
                stream_out(comm_ccw.at[ss], 0, V2, o_ccw * V + V2,
                           m_col, inv)
                stream_out(l_ref, 0, V, my * V, m_col, inv)

            dcw.wait()
            dccw.wait()
            if h == 1:
                pl.semaphore_signal(cw_credit, inc=1, device_id=(left,),
                                    device_id_type=pl.DeviceIdType.MESH)
                pl.semaphore_signal(ccw_credit, inc=1, device_id=(right,),
                                    device_id_type=pl.DeviceIdType.MESH)

        V4 = V2 // 2
        pl.semaphore_wait(cw_credit, 1)
        pl.semaphore_wait(ccw_credit, 1)
        d2 = []
        for commbuf, sends, recvs, tgt in (
                (comm_cw, cw_send, cw_recv, right),
                (comm_ccw, ccw_send, ccw_recv, left)):
            for sub, (si, ri) in enumerate(((0, 1), (2, 2))):
                d = pltpu.make_async_remote_copy(
                    src_ref=commbuf.at[0, :, pl.ds(sub * V4, V4)],
                    dst_ref=commbuf.at[1, :, pl.ds(sub * V4, V4)],
                    send_sem=sends.at[si], recv_sem=recvs.at[ri],
                    device_id=(tgt,), device_id_type=pl.DeviceIdType.MESH)
                d.start()
                d2.append(d)
        dcw_a, dcw_b, dccw_a, dccw_b = d2

        m_col, inv = stats_state["m"], stats_state["inv"]
        stream_out(comm_cw.at[0], 0, V2, ((my + 2) % N_DEV) * V, m_col, inv)
        stream_out(comm_ccw.at[0], 0, V2, ((my + 2) % N_DEV) * V + V2,
                   m_col, inv)

        o_cw = ((my + 1) % N_DEV) * V
        o_ccw = ((my + 3) % N_DEV) * V + V2
        dcw_a.wait()
        stream_out(comm_cw.at[1], 0, V4, o_cw, m_col, inv)
        dccw_a.wait()
        stream_out(comm_ccw.at[1], 0, V4, o_ccw, m_col, inv)
        dcw_b.wait()
        stream_out(comm_cw.at[1], V4, V4, o_cw + V4, m_col, inv)
        dccw_b.wait()
        stream_out(comm_ccw.at[1], V4, V4, o_ccw + V4, m_col, inv)
        for p in (0, 1):
            if pending[p] is not None:
                pending[p].wait()

    return pl.pallas_call(
        body,
        out_shape=jax.ShapeDtypeStruct((T, N_DEV * V), jnp.float32),
        in_specs=[pl.BlockSpec(memory_space=pltpu.VMEM)],
        out_specs=pl.BlockSpec(memory_space=pl.ANY),
        scratch_shapes=[
            pltpu.VMEM((2, T, V2), jnp.bfloat16),
            pltpu.VMEM((2, T, V2), jnp.bfloat16),
            pltpu.VMEM((2, T, CH), jnp.float32),
            pltpu.VMEM((N_DEV, T, 128), jnp.float32),
            pltpu.SemaphoreType.DMA((3,)),
            pltpu.SemaphoreType.DMA((3,)),
            pltpu.SemaphoreType.DMA((3,)),
            pltpu.SemaphoreType.DMA((3,)),
            pltpu.SemaphoreType.DMA((3,)),
            pltpu.SemaphoreType.DMA((3,)),
            pltpu.SemaphoreType.DMA((2,)),
            pltpu.SemaphoreType.REGULAR,
            pltpu.SemaphoreType.REGULAR,
        ],
        compiler_params=pltpu.CompilerParams(
            collective_id=0, vmem_limit_bytes=63 * 1024 * 1024),
    )(logits)


# device time: 219101 ns/iter; 1.7775x vs baseline; 1.0230x over previous
import jax
import jax.numpy as jnp
from jax import lax
from jax.experimental import pallas as pl
from jax.experimental.pallas import tpu as pltpu

N_DEV = 4


def kernel(x, W):
    T = x.shape[0]
    V = W.shape[1]
    V2 = V // 2
    CH = 1024
    logits = jnp.dot(x, W, preferred_element_type=jnp.bfloat16)

    def body(l_ref, out_ref, comm_cw, comm_ccw, stage, sbuf,
             cw_send, cw_recv, ccw_send, ccw_recv,
             s_send, s_recv, stage_sems, cw_credit, ccw_credit):
        my = lax.axis_index("i")
        left = (my + N_DEV - 1) % N_DEV
        right = (my + 1) % N_DEV
        diag = (my + 2) % N_DEV

        barrier = pltpu.get_barrier_semaphore()
        for nbr in (left, right, diag):
            pl.semaphore_signal(barrier, inc=1, device_id=(nbr,),
                                device_id_type=pl.DeviceIdType.MESH)
        pl.semaphore_wait(barrier, 3)

        pending = {0: None, 1: None}
        pctr = [0]

        def stream_out(src_ref, src_col, ncols, out_base, m_col, inv):
            for c in range(0, ncols, CH):
                p = pctr[0] % 2
                pctr[0] += 1
                if pending[p] is not None:
                    pending[p].wait()
                stage[p] = (jnp.exp(src_ref[:, src_col + c:src_col + c + CH]
                                    - m_col) * inv)
                cp = pltpu.make_async_copy(
                    stage.at[p], out_ref.at[:, pl.ds(out_base + c, CH)],
                    stage_sems.at[p])
                cp.start()
                pending[p] = cp

        stats_state = {}
        for h in range(2):
            ss, rs = h % 2, (h + 1) % 2
            dcw = pltpu.make_async_remote_copy(
                src_ref=l_ref.at[:, 0:V2] if h == 0 else comm_cw.at[ss],
                dst_ref=comm_cw.at[rs],
                send_sem=cw_send.at[ss], recv_sem=cw_recv.at[rs],
                device_id=(right,), device_id_type=pl.DeviceIdType.MESH)
            dccw = pltpu.make_async_remote_copy(
                src_ref=l_ref.at[:, V2:V] if h == 0 else comm_ccw.at[ss],
                dst_ref=comm_ccw.at[rs],
                send_sem=ccw_send.at[ss], recv_sem=ccw_recv.at[rs],
                device_id=(left,), device_id_type=pl.DeviceIdType.MESH)
            dcw.start()
            dccw.start()

            if h == 0:
                m_loc = jnp.max(
                    l_ref[:, 0:CH], axis=1, keepdims=True
                ).astype(jnp.float32)
                for c in range(CH, V, CH):
                    m_loc = jnp.maximum(m_loc, jnp.max(
                        l_ref[:, c:c + CH], axis=1, keepdims=True))
                s_loc = jnp.zeros((T, 1), jnp.float32)
                for c in range(0, V, CH):
                    s_loc = s_loc + jnp.sum(
                        jnp.exp(l_ref[:, c:c + CH] - m_loc),
                        axis=1, keepdims=True)
                mg = jnp.broadcast_to(m_loc, (T, 64))
                sg = jnp.broadcast_to(s_loc, (T, 64))
                sbuf[3] = jnp.concatenate([mg, sg], axis=1)

                stats_rdmas = []
                for slot, tgt in ((0, right), (1, left), (2, diag)):
                    d = pltpu.make_async_remote_copy(
                        src_ref=sbuf.at[3], dst_ref=sbuf.at[slot],
                        send_sem=s_send.at[slot], recv_sem=s_recv.at[slot],
                        device_id=(tgt,),
                        device_id_type=pl.DeviceIdType.MESH)
                    d.start()
                    stats_rdmas.append(d)
                for d in stats_rdmas:
                    d.wait()
                for slot in range(3):
                    rm = sbuf[slot, :, 0:64]
                    rsum = sbuf[slot, :, 64:128]
                    m_new = jnp.maximum(mg, rm)
                    sg = sg * jnp.exp(mg - m_new) + rsum * jnp.exp(rm - m_new)
                    mg = m_new
                m_col = mg[:, 0:1]
                inv = 1.0 / sg[:, 0:1]
                stats_state["m"] = m_col
                stats_state["inv"] = inv
            else:
                m_col, inv = stats_state["m"], stats_state["inv"]
                o_cw = (my + N_DEV - h) % N_DEV
                stream_out(comm_cw.at[ss], 0, V2, o_cw * V, m_col, inv)
                o_ccw = (my + h) % N_DEV
                stream_out(comm_ccw.at[ss], 0, V2, o_ccw * V + V2,
                           m_col, inv)
                stream_out(l_ref, 0, V, my * V, m_col, inv)

            dcw.wait()
            dccw.wait()
            if h == 1:
                pl.semaphore_signal(cw_credit, inc=1, device_id=(left,),
                                    device_id_type=pl.DeviceIdType.MESH)
                pl.semaphore_signal(ccw_credit, inc=1, device_id=(right,),
                                    device_id_type=pl.DeviceIdType.MESH)

        V4 = V2 // 2
        pl.semaphore_wait(cw_credit, 1)
        pl.semaphore_wait(ccw_credit, 1)
        d2 = []
        for commbuf, sends, recvs, tgt in (
                (comm_cw, cw_send, cw_recv, right),
                (comm_ccw, ccw_send, ccw_recv, left)):
            for sub, (si, ri) in enumerate(((0, 1), (2, 2))):
                d = pltpu.make_async_remote_copy(
                    src_ref=commbuf.at[0, :, pl.ds(sub * V4, V4)],
                    dst_ref=commbuf.at[1, :, pl.ds(sub * V4, V4)],
                    send_sem=sends.at[si], recv_sem=recvs.at[ri],
                    device_id=(tgt,), device_id_type=pl.DeviceIdType.MESH)
                d.start()
                d2.append(d)
        dcw_a, dcw_b, dccw_a, dccw_b = d2

        m_col, inv = stats_state["m"], stats_state["inv"]
        stream_out(comm_cw.at[0], 0, V2, ((my + 2) % N_DEV) * V, m_col, inv)
        stream_out(comm_ccw.at[0], 0, V2, ((my + 2) % N_DEV) * V + V2,
                   m_col, inv)

        o_cw = ((my + 1) % N_DEV) * V
        o_ccw = ((my + 3) % N_DEV) * V + V2
        dcw_a.wait()
        stream_out(comm_cw.at[1], 0, V4, o_cw, m_col, inv)
        dccw_a.wait()
        stream_out(comm_ccw.at[1], 0, V4, o_ccw, m_col, inv)
        dcw_b.wait()
        stream_out(comm_cw.at[1], V4, V4, o_cw + V4, m_col, inv)
        dccw_b.wait()
        stream_out(comm_ccw.at[1], V4, V4, o_ccw + V4, m_col, inv)
        for p in (0, 1):
            if pending[p] is not None:
                pending[p].wait()

    return pl.pallas_call(
        body,
        out_shape=jax.ShapeDtypeStruct((T, N_DEV * V), jnp.float32),
        in_specs=[pl.BlockSpec(memory_space=pltpu.VMEM)],
        out_specs=pl.BlockSpec(memory_space=pl.ANY),
        scratch_shapes=[
            pltpu.VMEM((2, T, V2), jnp.bfloat16),
            pltpu.VMEM((2, T, V2), jnp.bfloat16),
            pltpu.VMEM((2, T, CH), jnp.float32),
            pltpu.VMEM((N_DEV, T, 128), jnp.float32),
            pltpu.SemaphoreType.DMA((3,)),
            pltpu.SemaphoreType.DMA((3,)),
            pltpu.SemaphoreType.DMA((3,)),
            pltpu.SemaphoreType.DMA((3,)),
            pltpu.SemaphoreType.DMA((3,)),
            pltpu.SemaphoreType.DMA((3,)),
            pltpu.SemaphoreType.DMA((2,)),
            pltpu.SemaphoreType.REGULAR,
            pltpu.SemaphoreType.REGULAR,
        ],
        compiler_params=pltpu.CompilerParams(
            collective_id=0, vmem_limit_bytes=63 * 1024 * 1024),
    )(logits)
